# baseline (speedup 1.0000x reference)
import jax
import jax.numpy as jnp
from jax.experimental import pallas as pl
from jax.experimental.pallas import tpu as pltpu

_CK = 128


def _pos_kernel(row_ref, col_ref, out_ref, *rest):
    bufs, sem = rest[:-1], rest[-1]
    H, half = row_ref.shape
    W = col_ref.shape[0]
    HW = H * W
    B = out_ref.shape[0]
    n_half = half // _CK
    p = jax.lax.broadcasted_iota(jnp.int32, (H, HW), 1)
    i = jax.lax.broadcasted_iota(jnp.int32, (H, HW), 0)
    sel_row = (p // W == i).astype(jnp.float32)
    sel_col = (p % W == i).astype(jnp.float32)
    dn = (((0,), (0,)), ((), ()))
    copies = []
    for k in range(2 * n_half):
        if k < n_half:
            blk = row_ref[:, k * _CK:(k + 1) * _CK]
            sel = sel_row
        else:
            blk = col_ref[:, (k - n_half) * _CK:(k - n_half + 1) * _CK]
            sel = sel_col
        chunk = jax.lax.dot_general(blk, sel, dn,
                                    preferred_element_type=jnp.float32,
                                    precision=jax.lax.Precision.HIGHEST)
        buf = bufs[k]
        for b in range(B):
            buf[b] = chunk
        c = pltpu.make_async_copy(
            buf, out_ref.at[:, k * _CK:(k + 1) * _CK, :], sem.at[k])
        c.start()
        copies.append(c)
    for c in copies:
        c.wait()


def kernel(feat, row_embed, col_embed):
    B, C, H, W = feat.shape
    half = row_embed.shape[1]
    n_chunks = C // _CK
    out = pl.pallas_call(
        _pos_kernel,
        in_specs=[
            pl.BlockSpec(memory_space=pltpu.MemorySpace.VMEM),
            pl.BlockSpec(memory_space=pltpu.MemorySpace.VMEM),
        ],
        out_specs=pl.BlockSpec(memory_space=pltpu.MemorySpace.HBM),
        out_shape=jax.ShapeDtypeStruct((B, C, H * W), jnp.float32),
        scratch_shapes=(
            [pltpu.VMEM((B, _CK, H * W), jnp.float32) for _ in range(n_chunks)]
            + [pltpu.SemaphoreType.DMA((n_chunks,))]
        ),
    )(row_embed[:H], col_embed[:W])
    return out.reshape(B, C, H, W)
